# baseline (speedup 1.0000x reference)
import functools

import jax
import jax.numpy as jnp
from jax import lax
from jax.experimental import pallas as pl
from jax.experimental.pallas import tpu as pltpu
from jax.experimental.pallas import tpu_sc as plsc

F32 = jnp.float32
I32 = jnp.int32

_N = 10000
_NP = 10240
_E = 170000
_EP = 172032
_EPT = _EP // 32
_EB = 128
_NBLK = _EPT // _EB
_B = 5000
_BP = 5120
_D = 256
_HW = 128
_RPT = _NP // 16
_PNRT = _NP // 32
_PTRT = _BP // 16

_SC_MESH = dict(core_axis_name="c", subcore_axis_name="s")
_SC_PARAMS = pltpu.CompilerParams(needs_layout_passes=False)


def _emb_body(t_ref, a_ref, gf_ref, Wt, bt, Wa, ba, Wg, bg, x1o, x2o, go):
    x1o[...] = jnp.dot(t_ref[...], Wt[...], preferred_element_type=F32) + bt[...]
    x2o[...] = jnp.dot(a_ref[...], Wa[...], preferred_element_type=F32) + ba[...]
    go[...] = jnp.dot(gf_ref[...], Wg[...], preferred_element_type=F32) + bg[...]


def _embed(type_nodes, attr_nodes, global_features, Wt, bt, Wa, ba, Wg, bg):
    return pl.pallas_call(
        _emb_body,
        out_shape=[jax.ShapeDtypeStruct((5000, 128), F32)] * 3,
    )(type_nodes, attr_nodes, global_features,
      Wt, bt.reshape(1, -1), Wa, ba.reshape(1, -1), Wg, bg.reshape(1, -1))


def _split_xl(x, Wl, Wr, xr_o, xh0_o, xh1_o):
    xl = jnp.dot(x, Wl[...], preferred_element_type=F32)
    xr_o[...] = jnp.dot(x, Wr[...], preferred_element_type=F32)
    xh0_o[...] = xl[:, :_HW]
    xh1_o[...] = xl[:, _HW:]


def _mm1_body(x_ref, Wl, Wr, xr_o, xh0_o, xh1_o):
    _split_xl(x_ref[...], Wl, Wr, xr_o, xh0_o, xh1_o)


def _combine(p0_ref, p1_ref, dn_ref, b_ref):
    acc = jnp.concatenate([p0_ref[0] + p0_ref[1], p1_ref[0] + p1_ref[1]], axis=1)
    dsum = jnp.sum(dn_ref[...], axis=0, keepdims=True)
    denom = jnp.reshape(dsum, (dsum.shape[1], 1))
    h = acc / (denom + 1e-16) + b_ref[...]
    return jnp.where(h > 0, h, 0.01 * h)


def _mm23_body(p0_ref, p1_ref, dn_ref, b_ref, Wl, Wr, xr_o, xh0_o, xh1_o):
    _split_xl(_combine(p0_ref, p1_ref, dn_ref, b_ref), Wl, Wr, xr_o, xh0_o, xh1_o)


def _post3_body(p0_ref, p1_ref, dn_ref, b_ref, x_o0, x_o1):
    x = _combine(p0_ref, p1_ref, dn_ref, b_ref)
    x_o0[...] = x[:, :_HW]
    x_o1[...] = x[:, _HW:]


_R_MM = 1280
_G_MM = _NP // _R_MM

_MM_OUT_SPECS = [
    pl.BlockSpec((_R_MM, _D), lambda i: (i, 0)),
    pl.BlockSpec((_R_MM, _HW), lambda i: (i, 0)),
    pl.BlockSpec((_R_MM, _HW), lambda i: (i, 0)),
]
_MM_OUT_SHAPE = [
    jax.ShapeDtypeStruct((_NP, _D), F32),
    jax.ShapeDtypeStruct((_NP, _HW), F32),
    jax.ShapeDtypeStruct((_NP, _HW), F32),
]


def _mm1(x0, Wl, Wr):
    return pl.pallas_call(
        _mm1_body,
        grid=(_G_MM,),
        in_specs=[
            pl.BlockSpec((_R_MM, 128), lambda i: (i, 0)),
            pl.BlockSpec((128, _D), lambda i: (0, 0)),
            pl.BlockSpec((128, _D), lambda i: (0, 0)),
        ],
        out_specs=_MM_OUT_SPECS,
        out_shape=_MM_OUT_SHAPE,
    )(x0, Wl, Wr)


_P_SPECS = [
    pl.BlockSpec((2, _R_MM, _HW), lambda i: (0, i, 0)),
    pl.BlockSpec((2, _R_MM, _HW), lambda i: (0, i, 0)),
    pl.BlockSpec((32, _R_MM), lambda i: (0, i)),
    pl.BlockSpec((1, _D), lambda i: (0, 0)),
]


def _mm23(P0, P1, dn, b, Wl, Wr):
    return pl.pallas_call(
        _mm23_body,
        grid=(_G_MM,),
        in_specs=_P_SPECS + [
            pl.BlockSpec((_D, _D), lambda i: (0, 0)),
            pl.BlockSpec((_D, _D), lambda i: (0, 0)),
        ],
        out_specs=_MM_OUT_SPECS,
        out_shape=_MM_OUT_SHAPE,
    )(P0, P1, dn, b.reshape(1, -1), Wl, Wr)


def _post3(P0, P1, dn, b):
    return pl.pallas_call(
        _post3_body,
        grid=(_G_MM,),
        in_specs=_P_SPECS,
        out_specs=[pl.BlockSpec((_R_MM, _HW), lambda i: (i, 0))] * 2,
        out_shape=[jax.ShapeDtypeStruct((_NP, _HW), F32)] * 2,
    )(P0, P1, dn, b.reshape(1, -1))


def _head_body(pa_ref, pb_ref, cnt_ref, g_ref, lW, lb, sW, sb, emb_o, prob_o):
    acc = jnp.concatenate([pa_ref[0] + pa_ref[1], pb_ref[0] + pb_ref[1]], axis=1)
    csum = jnp.sum(cnt_ref[...], axis=0, keepdims=True)
    cnt = jnp.reshape(csum, (csum.shape[1], 1))
    pooled = acc / jnp.clip(cnt, 1.0, None)
    h = jnp.concatenate([pooled, g_ref[...]], axis=1)
    emb = jnp.dot(h, lW[...], preferred_element_type=F32) + lb[...]
    emb_o[...] = emb
    lg = jnp.dot(emb, sW[...], preferred_element_type=F32) + sb[...]
    m = jnp.max(lg, axis=1, keepdims=True)
    p = jnp.exp(lg - m)
    prob_o[...] = p / jnp.sum(p, axis=1, keepdims=True)


def _head(Pa, Pb, cnt, g, lW, lb, sWp, sbp):
    return pl.pallas_call(
        _head_body,
        grid=(4,),
        in_specs=[
            pl.BlockSpec((2, 1280, _HW), lambda i: (0, i, 0)),
            pl.BlockSpec((2, 1280, _HW), lambda i: (0, i, 0)),
            pl.BlockSpec((32, 1280), lambda i: (0, i)),
            pl.BlockSpec((1280, 128), lambda i: (i, 0)),
            pl.BlockSpec((384, 128), lambda i: (0, 0)),
            pl.BlockSpec((1, 128), lambda i: (0, 0)),
            pl.BlockSpec((128, 8), lambda i: (0, 0)),
            pl.BlockSpec((1, 8), lambda i: (0, 0)),
        ],
        out_specs=[
            pl.BlockSpec((1280, 128), lambda i: (i, 0)),
            pl.BlockSpec((1280, 8), lambda i: (i, 0)),
        ],
        out_shape=[
            jax.ShapeDtypeStruct((_BP, 128), F32),
            jax.ShapeDtypeStruct((_BP, 8), F32),
        ],
    )(Pa, Pb, cnt, g, lW, lb.reshape(1, -1), sWp, sbp)


_EBA = 64
_NBA = _EPT // _EBA
_EBB = 96
_NBB = _EPT // _EBB


def _passA_body(xl0, xl1, xr, src, dst, att, e_o, mx_o,
                src_p, dst_p, bufs, e_p, att_b, acc_b, mx_b, sems):
    cid = lax.axis_index("c")
    sid = lax.axis_index("s")
    wid = sid * 2 + cid
    pltpu.sync_copy(att, att_b)
    pltpu.sync_copy(src.at[pl.ds(wid * _EPT, _EPT)], src_p)
    pltpu.sync_copy(dst.at[pl.ds(wid * _EPT, _EPT)], dst_p)
    att_v = [att_b[pl.ds(16 * k, 16)] for k in range(16)]
    zero16 = jnp.zeros((16,), F32)
    lanes = lax.iota(I32, 16)
    col_idx = [jnp.full((16,), j, I32) for j in range(16)]

    def issue(off, s):
        r0, r1, r2 = bufs[s]
        t0, t1, t2 = sems[s]
        pltpu.async_copy(xl0.at[src_p.at[pl.ds(off, _EBA)]], r0, t0)
        pltpu.async_copy(xl1.at[src_p.at[pl.ds(off, _EBA)]], r1, t1)
        pltpu.async_copy(xr.at[dst_p.at[pl.ds(off, _EBA)]], r2, t2)

    def waitset(off, s):
        r0, r1, r2 = bufs[s]
        t0, t1, t2 = sems[s]
        pltpu.make_async_copy(xl0.at[src_p.at[pl.ds(off, _EBA)]], r0, t0).wait()
        pltpu.make_async_copy(xl1.at[src_p.at[pl.ds(off, _EBA)]], r1, t1).wait()
        pltpu.make_async_copy(xr.at[dst_p.at[pl.ds(off, _EBA)]], r2, t2).wait()

    def compute(off, s, mxv):
        r0, r1, r2 = bufs[s]

        def grp(gi, mxv1):
            def edge(j4, c):
                for jj in range(4):
                    j = j4 * 4 + jj
                    i = gi * 16 + j
                    acc = zero16
                    for k in range(8):
                        v = r0[i, pl.ds(16 * k, 16)] + r2[i, pl.ds(16 * k, 16)]
                        acc = acc + jnp.maximum(v, 0.2 * v) * att_v[k]
                    for k in range(8):
                        v = r1[i, pl.ds(16 * k, 16)] + r2[i, pl.ds(128 + 16 * k, 16)]
                        acc = acc + jnp.maximum(v, 0.2 * v) * att_v[8 + k]
                    acc_b[j, pl.ds(0, 16)] = acc
                return c

            lax.fori_loop(0, 4, edge, 0)
            e16 = plsc.load_gather(acc_b, [lanes, col_idx[0]])
            for j in range(1, 16):
                e16 = e16 + plsc.load_gather(acc_b, [lanes, col_idx[j]])
            e_p[pl.ds(off + gi * 16, 16)] = e16
            return jnp.maximum(mxv1, e16)

        return lax.fori_loop(0, _EBA // 16, grp, mxv)

    issue(0, 0)

    def outer(g, mxv):
        off_a = 2 * g * _EBA
        off_b = off_a + _EBA
        issue(off_b, 1)
        waitset(off_a, 0)
        mxv = compute(off_a, 0, mxv)

        @pl.when(2 * g + 2 < _NBA)
        def _():
            issue(off_a + 2 * _EBA, 0)

        waitset(off_b, 1)
        mxv = compute(off_b, 1, mxv)
        return mxv

    mxv = lax.fori_loop(0, _NBA // 2, outer, jnp.full((16,), F32(-3e38), F32))
    pltpu.sync_copy(e_p, e_o.at[pl.ds(wid * _EPT, _EPT)])
    mx_b[...] = mxv
    pltpu.sync_copy(mx_b, mx_o.at[wid])


@functools.partial(
    pl.kernel,
    out_type=(jax.ShapeDtypeStruct((_EP,), F32),
              jax.ShapeDtypeStruct((32, 16), F32)),
    mesh=plsc.VectorSubcoreMesh(**_SC_MESH),
    compiler_params=_SC_PARAMS,
    scratch_types=[
        pltpu.VMEM((_EPT,), I32),
        pltpu.VMEM((_EPT,), I32),
        [[pltpu.VMEM((_EBA, _HW), F32), pltpu.VMEM((_EBA, _HW), F32),
          pltpu.VMEM((_EBA, _D), F32)] for _ in range(2)],
        pltpu.VMEM((_EPT,), F32),
        pltpu.VMEM((_D,), F32),
        pltpu.VMEM((16, 16), F32),
        pltpu.VMEM((16,), F32),
        [[pltpu.SemaphoreType.DMA] * 3 for _ in range(2)],
    ],
)
def _passA(xl0, xl1, xr, src, dst, att, e_o, mx_o, *rest):
    _passA_body(xl0, xl1, xr, src, dst, att, e_o, mx_o, *rest)


def _passB_body(xh0, xh1, src, dst, e_in, mx_in, out0, out1, dn_o,
                src_s, dst_b, dst_v, e_cb, rows2, mxs, zb, stage, dtab,
                table, sems):
    cid = lax.axis_index("c")
    sid = lax.axis_index("s")
    wid = sid * 2 + cid
    base = wid * _EPT
    z16 = jnp.zeros((16,), F32)
    for r in range(16):
        for k in range(_HW // 16):
            zb[r, pl.ds(16 * k, 16)] = z16
    rbase = sid * _RPT

    def zero_table(j, carry):
        pltpu.sync_copy(zb, table.at[pl.ds(rbase + j * 16, 16)])
        return carry

    lax.fori_loop(0, _RPT // 16, zero_table, 0)

    def zdt(j, carry):
        dtab[pl.ds(j * 16, 16)] = z16
        return carry

    lax.fori_loop(0, _NP // 16, zdt, 0)
    plsc.subcore_barrier()

    pltpu.sync_copy(mx_in, mxs)
    m = mxs[0, :]
    for j in range(1, 32):
        m = jnp.maximum(m, mxs[j, :])
    M = jnp.max(m)
    lanes = lax.iota(I32, 16)
    zero16 = jnp.zeros((16,), F32)

    def half(xlt, with_denom):
        def issue(off, s):
            pltpu.sync_copy(src.at[pl.ds(base + off, _EBB)], src_s[s])
            pltpu.async_copy(xlt.at[src_s[s]], rows2[s], sems[s])

        def waitset(s):
            pltpu.make_async_copy(xlt.at[src_s[s]], rows2[s], sems[s]).wait()

        def consume(off, s, carry):
            rows = rows2[s]
            pltpu.sync_copy(dst.at[pl.ds(base + off, _EBB)], dst_b)
            if with_denom:
                pltpu.sync_copy(dst.at[pl.ds(base + off, _EBB)], dst_v)
            pltpu.sync_copy(e_in.at[pl.ds(base + off, _EBB)], e_cb)

            def grp(gi, c1):
                ee16 = jnp.exp(e_cb[pl.ds(16 * gi, 16)] - M)
                if with_denom:
                    dst16 = dst_v[pl.ds(16 * gi, 16)]
                    plsc.addupdate_scatter(dtab, [dst16], ee16)

                def edge(j, c2):
                    i = gi * 16 + j
                    s2 = jnp.sum(jnp.where(lanes == j, ee16, zero16))
                    for k in range(_HW // 16):
                        rows[i, pl.ds(16 * k, 16)] = rows[i, pl.ds(16 * k, 16)] * s2
                    return c2

                return lax.fori_loop(0, 16, edge, c1)

            lax.fori_loop(0, _EBB // 16, grp, 0)
            pltpu.async_copy(rows, table.at[dst_b], sems[s], add=True).wait()
            return carry

        issue(0, 0)

        def outer(g, carry):
            off_a = 2 * g * _EBB
            off_b = off_a + _EBB
            issue(off_b, 1)
            waitset(0)
            carry = consume(off_a, 0, carry)

            @pl.when(2 * g + 2 < _NBB)
            def _():
                issue(off_a + 2 * _EBB, 0)

            waitset(1)
            carry = consume(off_b, 1, carry)
            return carry

        lax.fori_loop(0, _NBB // 2, outer, 0)

    def dump(out):
        def step(j, carry):
            r0 = rbase + j * 32
            pltpu.sync_copy(table.at[pl.ds(r0, 32)], stage)
            pltpu.sync_copy(stage, out.at[cid, pl.ds(r0, 32)])
            return carry

        lax.fori_loop(0, _RPT // 32, step, 0)

    half(xh0, True)
    pltpu.sync_copy(dtab, dn_o.at[wid])
    plsc.subcore_barrier()
    dump(out0)
    lax.fori_loop(0, _RPT // 16, zero_table, 0)
    plsc.subcore_barrier()
    half(xh1, False)
    plsc.subcore_barrier()
    dump(out1)


_passB = pl.kernel(
    _passB_body,
    out_type=(jax.ShapeDtypeStruct((2, _NP, _HW), F32),
              jax.ShapeDtypeStruct((2, _NP, _HW), F32),
              jax.ShapeDtypeStruct((32, _NP), F32)),
    mesh=plsc.VectorSubcoreMesh(**_SC_MESH),
    compiler_params=_SC_PARAMS,
    scratch_types=[
        [pltpu.VMEM((_EBB,), I32) for _ in range(2)],
        pltpu.VMEM((_EBB,), I32),
        pltpu.VMEM((_EBB,), I32),
        pltpu.VMEM((_EBB,), F32),
        [pltpu.VMEM((_EBB, _HW), F32) for _ in range(2)],
        pltpu.VMEM((32, 16), F32),
        pltpu.VMEM((16, _HW), F32),
        pltpu.VMEM((32, _HW), F32),
        pltpu.VMEM((_NP,), F32),
        pltpu.VMEM_SHARED((_NP, _HW), F32),
        [pltpu.SemaphoreType.DMA for _ in range(2)],
    ],
)


def _pool_body(x30, x31, pidx, out_a, out_b, cnt_o,
               buf_a, buf_b, idx_b, idx_v, zb, stage, ctab, tab_a, tab_b,
               sem_a, sem_b):
    cid = lax.axis_index("c")
    sid = lax.axis_index("s")
    wid = sid * 2 + cid
    z16 = jnp.zeros((16,), F32)
    ones16 = jnp.ones((16,), F32)
    for r in range(16):
        for k in range(_HW // 16):
            zb[r, pl.ds(16 * k, 16)] = z16
    tbase = sid * _PTRT

    def zloop(j, carry):
        pltpu.sync_copy(zb, tab_a.at[pl.ds(tbase + j * 16, 16)])
        pltpu.sync_copy(zb, tab_b.at[pl.ds(tbase + j * 16, 16)])
        return carry

    lax.fori_loop(0, _PTRT // 16, zloop, 0)

    def zct(j, carry):
        ctab[pl.ds(j * 16, 16)] = z16
        return carry

    lax.fori_loop(0, _BP // 16, zct, 0)
    plsc.subcore_barrier()

    nbase = wid * _PNRT

    def blk(j, carry):
        r0 = nbase + j * 64
        pltpu.sync_copy(x30.at[pl.ds(r0, 64)], buf_a)
        pltpu.sync_copy(x31.at[pl.ds(r0, 64)], buf_b)
        pltpu.sync_copy(pidx.at[pl.ds(r0, 64)], idx_b)
        pltpu.sync_copy(pidx.at[pl.ds(r0, 64)], idx_v)
        for gi in range(4):
            plsc.addupdate_scatter(ctab, [idx_v[pl.ds(16 * gi, 16)]], ones16)
        ca = pltpu.async_copy(buf_a, tab_a.at[idx_b], sem_a, add=True)
        cb = pltpu.async_copy(buf_b, tab_b.at[idx_b], sem_b, add=True)
        ca.wait()
        cb.wait()
        return carry

    lax.fori_loop(0, _PNRT // 64, blk, 0)
    pltpu.sync_copy(ctab, cnt_o.at[wid])
    plsc.subcore_barrier()

    def dump(j, carry):
        r0 = tbase + j * 64
        pltpu.sync_copy(tab_a.at[pl.ds(r0, 64)], stage)
        pltpu.sync_copy(stage, out_a.at[cid, pl.ds(r0, 64)])
        pltpu.sync_copy(tab_b.at[pl.ds(r0, 64)], stage)
        pltpu.sync_copy(stage, out_b.at[cid, pl.ds(r0, 64)])
        return carry

    lax.fori_loop(0, _PTRT // 64, dump, 0)


_pool = pl.kernel(
    _pool_body,
    out_type=(jax.ShapeDtypeStruct((2, _BP, _HW), F32),
              jax.ShapeDtypeStruct((2, _BP, _HW), F32),
              jax.ShapeDtypeStruct((32, _BP), F32)),
    mesh=plsc.VectorSubcoreMesh(**_SC_MESH),
    compiler_params=_SC_PARAMS,
    scratch_types=[
        pltpu.VMEM((64, _HW), F32),
        pltpu.VMEM((64, _HW), F32),
        pltpu.VMEM((64,), I32),
        pltpu.VMEM((64,), I32),
        pltpu.VMEM((16, _HW), F32),
        pltpu.VMEM((64, _HW), F32),
        pltpu.VMEM((_BP,), F32),
        pltpu.VMEM_SHARED((_BP, _HW), F32),
        pltpu.VMEM_SHARED((_BP, _HW), F32),
        pltpu.SemaphoreType.DMA,
        pltpu.SemaphoreType.DMA,
    ],
)


def kernel(type_nodes, attr_nodes, edge_index, n_type_nodes, n_attr_nodes,
           global_features, batch_info, type_emb_W, type_emb_b, attr_emb_W,
           attr_emb_b, global_emb_W, global_emb_b, conv1_Wl, conv1_Wr,
           conv1_att, conv1_b, conv2_Wl, conv2_Wr, conv2_att, conv2_b,
           conv3_Wl, conv3_Wr, conv3_att, conv3_b, linear_W, linear_b,
           sm_W, sm_b):
    loop = jnp.arange(_N, dtype=edge_index.dtype)
    src = jnp.concatenate([edge_index[0], loop])
    dst = jnp.concatenate([edge_index[1], loop])
    zero1 = jnp.zeros((1,), n_type_nodes.dtype)
    t_off = jnp.concatenate([zero1, jnp.cumsum(n_type_nodes)[:-1]])
    a_off = 5000 + jnp.concatenate([zero1, jnp.cumsum(n_attr_nodes)[:-1]])
    gid = jnp.stack([t_off, a_off], axis=1).reshape(-1).astype(I32)
    src_p = jnp.concatenate([gid[src], jnp.zeros((_EP - _E,), I32)])
    dst_p = jnp.concatenate([gid[dst], jnp.full((_EP - _E,), _N, I32)])
    inv_gid = jnp.zeros((_N,), I32).at[gid].set(jnp.arange(_N, dtype=I32))
    pool_idx = jnp.concatenate([batch_info[inv_gid].astype(I32),
                                jnp.full((_NP - _N,), _B, I32)])

    x1, x2, g = _embed(type_nodes, attr_nodes, global_features,
                       type_emb_W, type_emb_b, attr_emb_W, attr_emb_b,
                       global_emb_W, global_emb_b)
    x0 = jnp.concatenate([x1, x2, jnp.zeros((_NP - _N, 128), F32)], axis=0)

    xr, xh0, xh1 = _mm1(x0, conv1_Wl, conv1_Wr)
    e1, mx1 = _passA(xh0, xh1, xr, src_p, dst_p, conv1_att)
    P10, P11, dn1 = _passB(xh0, xh1, src_p, dst_p, e1, mx1)

    xr, xh0, xh1 = _mm23(P10, P11, dn1, conv1_b, conv2_Wl, conv2_Wr)
    e2, mx2 = _passA(xh0, xh1, xr, src_p, dst_p, conv2_att)
    P20, P21, dn2 = _passB(xh0, xh1, src_p, dst_p, e2, mx2)

    xr, xh0, xh1 = _mm23(P20, P21, dn2, conv2_b, conv3_Wl, conv3_Wr)
    e3, mx3 = _passA(xh0, xh1, xr, src_p, dst_p, conv3_att)
    P30, P31, dn3 = _passB(xh0, xh1, src_p, dst_p, e3, mx3)

    x30, x31 = _post3(P30, P31, dn3, conv3_b)
    Pa, Pb, cnt = _pool(x30, x31, pool_idx)

    sWp = jnp.pad(sm_W, ((0, 0), (0, 2)))
    sbp = jnp.concatenate([sm_b, jnp.full((2,), -1e30, F32)]).reshape(1, -1)
    gp = jnp.concatenate([g, jnp.zeros((_BP - _B, 128), F32)], axis=0)
    emb, prob = _head(Pa, Pb, cnt, gp, linear_W, linear_b, sWp, sbp)
    return (prob[:_B, :6], emb[:_B])


# speedup vs baseline: 1.0485x; 1.0485x over previous
import functools

import jax
import jax.numpy as jnp
from jax import lax
from jax.experimental import pallas as pl
from jax.experimental.pallas import tpu as pltpu
from jax.experimental.pallas import tpu_sc as plsc

F32 = jnp.float32
I32 = jnp.int32

_N = 10000
_NP = 10240
_E = 170000
_EP = 172032
_EPT = _EP // 32
_EB = 128
_NBLK = _EPT // _EB
_B = 5000
_BP = 5120
_D = 256
_HW = 128
_RPT = _NP // 16
_PNRT = _NP // 32
_PTRT = _BP // 16

_SC_MESH = dict(core_axis_name="c", subcore_axis_name="s")
_SC_PARAMS = pltpu.CompilerParams(needs_layout_passes=False)


def _emb_body(t_ref, a_ref, gf_ref, Wt, bt, Wa, ba, Wg, bg, x1o, x2o, go):
    x1o[...] = jnp.dot(t_ref[...], Wt[...], preferred_element_type=F32) + bt[...]
    x2o[...] = jnp.dot(a_ref[...], Wa[...], preferred_element_type=F32) + ba[...]
    go[...] = jnp.dot(gf_ref[...], Wg[...], preferred_element_type=F32) + bg[...]


def _embed(type_nodes, attr_nodes, global_features, Wt, bt, Wa, ba, Wg, bg):
    return pl.pallas_call(
        _emb_body,
        out_shape=[jax.ShapeDtypeStruct((5000, 128), F32)] * 3,
    )(type_nodes, attr_nodes, global_features,
      Wt, bt.reshape(1, -1), Wa, ba.reshape(1, -1), Wg, bg.reshape(1, -1))


def _split_xl(x, Wl, Wr, xr_o, xh0_o, xh1_o):
    xl = jnp.dot(x, Wl[...], preferred_element_type=F32)
    xr_o[...] = jnp.dot(x, Wr[...], preferred_element_type=F32)
    xh0_o[...] = xl[:, :_HW]
    xh1_o[...] = xl[:, _HW:]


def _mm1_body(x_ref, Wl, Wr, xr_o, xh0_o, xh1_o):
    _split_xl(x_ref[...], Wl, Wr, xr_o, xh0_o, xh1_o)


def _combine(p0_ref, p1_ref, dn_ref, b_ref):
    acc = jnp.concatenate([p0_ref[0] + p0_ref[1], p1_ref[0] + p1_ref[1]], axis=1)
    dsum = jnp.sum(dn_ref[...], axis=0, keepdims=True)
    denom = jnp.reshape(dsum, (dsum.shape[1], 1))
    h = acc / (denom + 1e-16) + b_ref[...]
    return jnp.where(h > 0, h, 0.01 * h)


def _mm23_body(p0_ref, p1_ref, dn_ref, b_ref, Wl, Wr, xr_o, xh0_o, xh1_o):
    _split_xl(_combine(p0_ref, p1_ref, dn_ref, b_ref), Wl, Wr, xr_o, xh0_o, xh1_o)


def _post3_body(p0_ref, p1_ref, dn_ref, b_ref, x_o0, x_o1):
    x = _combine(p0_ref, p1_ref, dn_ref, b_ref)
    x_o0[...] = x[:, :_HW]
    x_o1[...] = x[:, _HW:]


_R_MM = 1280
_G_MM = _NP // _R_MM

_MM_OUT_SPECS = [
    pl.BlockSpec((_R_MM, _D), lambda i: (i, 0)),
    pl.BlockSpec((_R_MM, _HW), lambda i: (i, 0)),
    pl.BlockSpec((_R_MM, _HW), lambda i: (i, 0)),
]
_MM_OUT_SHAPE = [
    jax.ShapeDtypeStruct((_NP, _D), F32),
    jax.ShapeDtypeStruct((_NP, _HW), F32),
    jax.ShapeDtypeStruct((_NP, _HW), F32),
]


def _mm1(x0, Wl, Wr):
    return pl.pallas_call(
        _mm1_body,
        grid=(_G_MM,),
        in_specs=[
            pl.BlockSpec((_R_MM, 128), lambda i: (i, 0)),
            pl.BlockSpec((128, _D), lambda i: (0, 0)),
            pl.BlockSpec((128, _D), lambda i: (0, 0)),
        ],
        out_specs=_MM_OUT_SPECS,
        out_shape=_MM_OUT_SHAPE,
    )(x0, Wl, Wr)


_P_SPECS = [
    pl.BlockSpec((2, _R_MM, _HW), lambda i: (0, i, 0)),
    pl.BlockSpec((2, _R_MM, _HW), lambda i: (0, i, 0)),
    pl.BlockSpec((32, _R_MM), lambda i: (0, i)),
    pl.BlockSpec((1, _D), lambda i: (0, 0)),
]


def _mm23(P0, P1, dn, b, Wl, Wr):
    return pl.pallas_call(
        _mm23_body,
        grid=(_G_MM,),
        in_specs=_P_SPECS + [
            pl.BlockSpec((_D, _D), lambda i: (0, 0)),
            pl.BlockSpec((_D, _D), lambda i: (0, 0)),
        ],
        out_specs=_MM_OUT_SPECS,
        out_shape=_MM_OUT_SHAPE,
    )(P0, P1, dn, b.reshape(1, -1), Wl, Wr)


def _post3(P0, P1, dn, b):
    return pl.pallas_call(
        _post3_body,
        grid=(_G_MM,),
        in_specs=_P_SPECS,
        out_specs=[pl.BlockSpec((_R_MM, _HW), lambda i: (i, 0))] * 2,
        out_shape=[jax.ShapeDtypeStruct((_NP, _HW), F32)] * 2,
    )(P0, P1, dn, b.reshape(1, -1))


def _head_body(pa_ref, pb_ref, cnt_ref, g_ref, lW, lb, sW, sb, emb_o, prob_o):
    acc = jnp.concatenate([pa_ref[0] + pa_ref[1], pb_ref[0] + pb_ref[1]], axis=1)
    csum = jnp.sum(cnt_ref[...], axis=0, keepdims=True)
    cnt = jnp.reshape(csum, (csum.shape[1], 1))
    pooled = acc / jnp.clip(cnt, 1.0, None)
    h = jnp.concatenate([pooled, g_ref[...]], axis=1)
    emb = jnp.dot(h, lW[...], preferred_element_type=F32) + lb[...]
    emb_o[...] = emb
    lg = jnp.dot(emb, sW[...], preferred_element_type=F32) + sb[...]
    m = jnp.max(lg, axis=1, keepdims=True)
    p = jnp.exp(lg - m)
    prob_o[...] = p / jnp.sum(p, axis=1, keepdims=True)


def _head(Pa, Pb, cnt, g, lW, lb, sWp, sbp):
    return pl.pallas_call(
        _head_body,
        grid=(4,),
        in_specs=[
            pl.BlockSpec((2, 1280, _HW), lambda i: (0, i, 0)),
            pl.BlockSpec((2, 1280, _HW), lambda i: (0, i, 0)),
            pl.BlockSpec((32, 1280), lambda i: (0, i)),
            pl.BlockSpec((1280, 128), lambda i: (i, 0)),
            pl.BlockSpec((384, 128), lambda i: (0, 0)),
            pl.BlockSpec((1, 128), lambda i: (0, 0)),
            pl.BlockSpec((128, 8), lambda i: (0, 0)),
            pl.BlockSpec((1, 8), lambda i: (0, 0)),
        ],
        out_specs=[
            pl.BlockSpec((1280, 128), lambda i: (i, 0)),
            pl.BlockSpec((1280, 8), lambda i: (i, 0)),
        ],
        out_shape=[
            jax.ShapeDtypeStruct((_BP, 128), F32),
            jax.ShapeDtypeStruct((_BP, 8), F32),
        ],
    )(Pa, Pb, cnt, g, lW, lb.reshape(1, -1), sWp, sbp)


_EBA = 96
_NBA = _EPT // _EBA
_EBB = 96
_NBB = _EPT // _EBB


def _passA_body(xl0, xl1, xr, src, dst, att, e_o, mx_o,
                src_p, dst_p, bufs, e_p, att_b, acc_b, mx_b, sems):
    cid = lax.axis_index("c")
    sid = lax.axis_index("s")
    wid = sid * 2 + cid
    pltpu.sync_copy(att, att_b)
    pltpu.sync_copy(src.at[pl.ds(wid * _EPT, _EPT)], src_p)
    pltpu.sync_copy(dst.at[pl.ds(wid * _EPT, _EPT)], dst_p)
    att_v = [att_b[pl.ds(16 * k, 16)] for k in range(16)]
    zero16 = jnp.zeros((16,), F32)
    lanes = lax.iota(I32, 16)
    col_idx = [jnp.full((16,), j, I32) for j in range(16)]

    def issue(off, s):
        r0, r1, r2 = bufs[s]
        t0, t1, t2 = sems[s]
        pltpu.async_copy(xl0.at[src_p.at[pl.ds(off, _EBA)]], r0, t0)
        pltpu.async_copy(xl1.at[src_p.at[pl.ds(off, _EBA)]], r1, t1)
        pltpu.async_copy(xr.at[dst_p.at[pl.ds(off, _EBA)]], r2, t2)

    def waitset(off, s):
        r0, r1, r2 = bufs[s]
        t0, t1, t2 = sems[s]
        pltpu.make_async_copy(xl0.at[src_p.at[pl.ds(off, _EBA)]], r0, t0).wait()
        pltpu.make_async_copy(xl1.at[src_p.at[pl.ds(off, _EBA)]], r1, t1).wait()
        pltpu.make_async_copy(xr.at[dst_p.at[pl.ds(off, _EBA)]], r2, t2).wait()

    def compute(off, s, mxv):
        r0, r1, r2 = bufs[s]

        def grp(gi, mxv1):
            def edge(j, c):
                i = gi * 16 + j
                acc = zero16
                for k in range(8):
                    v = r0[i, pl.ds(16 * k, 16)] + r2[i, pl.ds(16 * k, 16)]
                    acc = acc + jnp.maximum(v, 0.2 * v) * att_v[k]
                for k in range(8):
                    v = r1[i, pl.ds(16 * k, 16)] + r2[i, pl.ds(128 + 16 * k, 16)]
                    acc = acc + jnp.maximum(v, 0.2 * v) * att_v[8 + k]
                acc_b[j, pl.ds(0, 16)] = acc
                return c

            lax.fori_loop(0, 16, edge, 0)
            e16 = plsc.load_gather(acc_b, [lanes, col_idx[0]])
            for j in range(1, 16):
                e16 = e16 + plsc.load_gather(acc_b, [lanes, col_idx[j]])
            e_p[pl.ds(off + gi * 16, 16)] = e16
            return jnp.maximum(mxv1, e16)

        return lax.fori_loop(0, _EBA // 16, grp, mxv)

    issue(0, 0)

    def outer(g, mxv):
        off_a = 2 * g * _EBA
        off_b = off_a + _EBA
        issue(off_b, 1)
        waitset(off_a, 0)
        mxv = compute(off_a, 0, mxv)

        @pl.when(2 * g + 2 < _NBA)
        def _():
            issue(off_a + 2 * _EBA, 0)

        waitset(off_b, 1)
        mxv = compute(off_b, 1, mxv)
        return mxv

    mxv = lax.fori_loop(0, _NBA // 2, outer, jnp.full((16,), F32(-3e38), F32))
    pltpu.sync_copy(e_p, e_o.at[pl.ds(wid * _EPT, _EPT)])
    mx_b[...] = mxv
    pltpu.sync_copy(mx_b, mx_o.at[wid])


@functools.partial(
    pl.kernel,
    out_type=(jax.ShapeDtypeStruct((_EP,), F32),
              jax.ShapeDtypeStruct((32, 16), F32)),
    mesh=plsc.VectorSubcoreMesh(**_SC_MESH),
    compiler_params=_SC_PARAMS,
    scratch_types=[
        pltpu.VMEM((_EPT,), I32),
        pltpu.VMEM((_EPT,), I32),
        [[pltpu.VMEM((_EBA, _HW), F32), pltpu.VMEM((_EBA, _HW), F32),
          pltpu.VMEM((_EBA, _D), F32)] for _ in range(2)],
        pltpu.VMEM((_EPT,), F32),
        pltpu.VMEM((_D,), F32),
        pltpu.VMEM((16, 16), F32),
        pltpu.VMEM((16,), F32),
        [[pltpu.SemaphoreType.DMA] * 3 for _ in range(2)],
    ],
)
def _passA(xl0, xl1, xr, src, dst, att, e_o, mx_o, *rest):
    _passA_body(xl0, xl1, xr, src, dst, att, e_o, mx_o, *rest)


def _make_passB(with_denom):
    def body(xlt, src, dst, e_in, mx_in, *rest):
        if with_denom:
            (out, dn_o, src_s, dst_b, dst_v, e_cb, rows2, mxs, zb, stage, dtab,
             table, sems) = rest
        else:
            (out, src_s, dst_b, dst_v, e_cb, rows2, mxs, zb, stage, table,
             sems) = rest
        cid = lax.axis_index("c")
        sid = lax.axis_index("s")
        wid = sid * 2 + cid
        base = wid * _EPT
        z16 = jnp.zeros((16,), F32)
        for r in range(16):
            for k in range(_HW // 16):
                zb[r, pl.ds(16 * k, 16)] = z16
        rbase = sid * _RPT

        def zloop(j, carry):
            pltpu.sync_copy(zb, table.at[pl.ds(rbase + j * 16, 16)])
            return carry

        lax.fori_loop(0, _RPT // 16, zloop, 0)
        if with_denom:
            def zdt(j, carry):
                dtab[pl.ds(j * 16, 16)] = z16
                return carry

            lax.fori_loop(0, _NP // 16, zdt, 0)
        plsc.subcore_barrier()

        pltpu.sync_copy(mx_in, mxs)
        m = mxs[0, :]
        for j in range(1, 32):
            m = jnp.maximum(m, mxs[j, :])
        M = jnp.max(m)
        lanes = lax.iota(I32, 16)
        zero16 = jnp.zeros((16,), F32)

        def issue(off, s):
            pltpu.sync_copy(src.at[pl.ds(base + off, _EBB)], src_s[s])
            pltpu.async_copy(xlt.at[src_s[s]], rows2[s], sems[s])

        def waitset(s):
            pltpu.make_async_copy(xlt.at[src_s[s]], rows2[s], sems[s]).wait()

        def consume(off, s, carry):
            rows = rows2[s]
            pltpu.sync_copy(dst.at[pl.ds(base + off, _EBB)], dst_b)
            if with_denom:
                pltpu.sync_copy(dst.at[pl.ds(base + off, _EBB)], dst_v)
            pltpu.sync_copy(e_in.at[pl.ds(base + off, _EBB)], e_cb)

            def grp(gi, c1):
                ee16 = jnp.exp(e_cb[pl.ds(16 * gi, 16)] - M)
                if with_denom:
                    dst16 = dst_v[pl.ds(16 * gi, 16)]
                    plsc.addupdate_scatter(dtab, [dst16], ee16)

                def edge(j, c2):
                    i = gi * 16 + j
                    s2 = jnp.sum(jnp.where(lanes == j, ee16, zero16))
                    for k in range(_HW // 16):
                        rows[i, pl.ds(16 * k, 16)] = rows[i, pl.ds(16 * k, 16)] * s2
                    return c2

                return lax.fori_loop(0, 16, edge, c1)

            lax.fori_loop(0, _EBB // 16, grp, 0)
            pltpu.async_copy(rows, table.at[dst_b], sems[s], add=True).wait()
            return carry

        issue(0, 0)

        def outer(g, carry):
            off_a = 2 * g * _EBB
            off_b = off_a + _EBB
            issue(off_b, 1)
            waitset(0)
            carry = consume(off_a, 0, carry)

            @pl.when(2 * g + 2 < _NBB)
            def _():
                issue(off_a + 2 * _EBB, 0)

            waitset(1)
            carry = consume(off_b, 1, carry)
            return carry

        lax.fori_loop(0, _NBB // 2, outer, 0)
        if with_denom:
            pltpu.sync_copy(dtab, dn_o.at[wid])
        plsc.subcore_barrier()

        def dump(j, carry):
            r0 = rbase + j * 32
            pltpu.sync_copy(table.at[pl.ds(r0, 32)], stage)
            pltpu.sync_copy(stage, out.at[cid, pl.ds(r0, 32)])
            return carry

        lax.fori_loop(0, _RPT // 32, dump, 0)

    out_type = [jax.ShapeDtypeStruct((2, _NP, _HW), F32)]
    scratch = [
        [pltpu.VMEM((_EBB,), I32) for _ in range(2)],
        pltpu.VMEM((_EBB,), I32),
        pltpu.VMEM((_EBB,), I32),
        pltpu.VMEM((_EBB,), F32),
        [pltpu.VMEM((_EBB, _HW), F32) for _ in range(2)],
        pltpu.VMEM((32, 16), F32),
        pltpu.VMEM((16, _HW), F32),
        pltpu.VMEM((32, _HW), F32),
    ]
    if with_denom:
        out_type.append(jax.ShapeDtypeStruct((32, _NP), F32))
        scratch.append(pltpu.VMEM((_NP,), F32))
    scratch += [
        pltpu.VMEM_SHARED((_NP, _HW), F32),
        [pltpu.SemaphoreType.DMA for _ in range(2)],
    ]
    return pl.kernel(
        body,
        out_type=tuple(out_type),
        mesh=plsc.VectorSubcoreMesh(**_SC_MESH),
        compiler_params=_SC_PARAMS,
        scratch_types=scratch,
    )


_passB0 = _make_passB(True)
_passB1 = _make_passB(False)


def _pool_body(x30, x31, pidx, out_a, out_b, cnt_o,
               buf_a, buf_b, idx_b, idx_v, zb, stage, ctab, tab_a, tab_b,
               sem_a, sem_b):
    cid = lax.axis_index("c")
    sid = lax.axis_index("s")
    wid = sid * 2 + cid
    z16 = jnp.zeros((16,), F32)
    ones16 = jnp.ones((16,), F32)
    for r in range(16):
        for k in range(_HW // 16):
            zb[r, pl.ds(16 * k, 16)] = z16
    tbase = sid * _PTRT

    def zloop(j, carry):
        pltpu.sync_copy(zb, tab_a.at[pl.ds(tbase + j * 16, 16)])
        pltpu.sync_copy(zb, tab_b.at[pl.ds(tbase + j * 16, 16)])
        return carry

    lax.fori_loop(0, _PTRT // 16, zloop, 0)

    def zct(j, carry):
        ctab[pl.ds(j * 16, 16)] = z16
        return carry

    lax.fori_loop(0, _BP // 16, zct, 0)
    plsc.subcore_barrier()

    nbase = wid * _PNRT

    def blk(j, carry):
        r0 = nbase + j * 64
        pltpu.sync_copy(x30.at[pl.ds(r0, 64)], buf_a)
        pltpu.sync_copy(x31.at[pl.ds(r0, 64)], buf_b)
        pltpu.sync_copy(pidx.at[pl.ds(r0, 64)], idx_b)
        pltpu.sync_copy(pidx.at[pl.ds(r0, 64)], idx_v)
        for gi in range(4):
            plsc.addupdate_scatter(ctab, [idx_v[pl.ds(16 * gi, 16)]], ones16)
        ca = pltpu.async_copy(buf_a, tab_a.at[idx_b], sem_a, add=True)
        cb = pltpu.async_copy(buf_b, tab_b.at[idx_b], sem_b, add=True)
        ca.wait()
        cb.wait()
        return carry

    lax.fori_loop(0, _PNRT // 64, blk, 0)
    pltpu.sync_copy(ctab, cnt_o.at[wid])
    plsc.subcore_barrier()

    def dump(j, carry):
        r0 = tbase + j * 64
        pltpu.sync_copy(tab_a.at[pl.ds(r0, 64)], stage)
        pltpu.sync_copy(stage, out_a.at[cid, pl.ds(r0, 64)])
        pltpu.sync_copy(tab_b.at[pl.ds(r0, 64)], stage)
        pltpu.sync_copy(stage, out_b.at[cid, pl.ds(r0, 64)])
        return carry

    lax.fori_loop(0, _PTRT // 64, dump, 0)


_pool = pl.kernel(
    _pool_body,
    out_type=(jax.ShapeDtypeStruct((2, _BP, _HW), F32),
              jax.ShapeDtypeStruct((2, _BP, _HW), F32),
              jax.ShapeDtypeStruct((32, _BP), F32)),
    mesh=plsc.VectorSubcoreMesh(**_SC_MESH),
    compiler_params=_SC_PARAMS,
    scratch_types=[
        pltpu.VMEM((64, _HW), F32),
        pltpu.VMEM((64, _HW), F32),
        pltpu.VMEM((64,), I32),
        pltpu.VMEM((64,), I32),
        pltpu.VMEM((16, _HW), F32),
        pltpu.VMEM((64, _HW), F32),
        pltpu.VMEM((_BP,), F32),
        pltpu.VMEM_SHARED((_BP, _HW), F32),
        pltpu.VMEM_SHARED((_BP, _HW), F32),
        pltpu.SemaphoreType.DMA,
        pltpu.SemaphoreType.DMA,
    ],
)


def kernel(type_nodes, attr_nodes, edge_index, n_type_nodes, n_attr_nodes,
           global_features, batch_info, type_emb_W, type_emb_b, attr_emb_W,
           attr_emb_b, global_emb_W, global_emb_b, conv1_Wl, conv1_Wr,
           conv1_att, conv1_b, conv2_Wl, conv2_Wr, conv2_att, conv2_b,
           conv3_Wl, conv3_Wr, conv3_att, conv3_b, linear_W, linear_b,
           sm_W, sm_b):
    loop = jnp.arange(_N, dtype=edge_index.dtype)
    src = jnp.concatenate([edge_index[0], loop])
    dst = jnp.concatenate([edge_index[1], loop])
    zero1 = jnp.zeros((1,), n_type_nodes.dtype)
    t_off = jnp.concatenate([zero1, jnp.cumsum(n_type_nodes)[:-1]])
    a_off = 5000 + jnp.concatenate([zero1, jnp.cumsum(n_attr_nodes)[:-1]])
    gid = jnp.stack([t_off, a_off], axis=1).reshape(-1).astype(I32)
    src_p = jnp.concatenate([gid[src], jnp.zeros((_EP - _E,), I32)])
    dst_p = jnp.concatenate([gid[dst], jnp.full((_EP - _E,), _N, I32)])
    inv_gid = jnp.zeros((_N,), I32).at[gid].set(jnp.arange(_N, dtype=I32))
    pool_idx = jnp.concatenate([batch_info[inv_gid].astype(I32),
                                jnp.full((_NP - _N,), _B, I32)])

    x1, x2, g = _embed(type_nodes, attr_nodes, global_features,
                       type_emb_W, type_emb_b, attr_emb_W, attr_emb_b,
                       global_emb_W, global_emb_b)
    x0 = jnp.concatenate([x1, x2, jnp.zeros((_NP - _N, 128), F32)], axis=0)

    xr, xh0, xh1 = _mm1(x0, conv1_Wl, conv1_Wr)
    e1, mx1 = _passA(xh0, xh1, xr, src_p, dst_p, conv1_att)
    P10, dn1 = _passB0(xh0, src_p, dst_p, e1, mx1)
    (P11,) = _passB1(xh1, src_p, dst_p, e1, mx1)

    xr, xh0, xh1 = _mm23(P10, P11, dn1, conv1_b, conv2_Wl, conv2_Wr)
    e2, mx2 = _passA(xh0, xh1, xr, src_p, dst_p, conv2_att)
    P20, dn2 = _passB0(xh0, src_p, dst_p, e2, mx2)
    (P21,) = _passB1(xh1, src_p, dst_p, e2, mx2)

    xr, xh0, xh1 = _mm23(P20, P21, dn2, conv2_b, conv3_Wl, conv3_Wr)
    e3, mx3 = _passA(xh0, xh1, xr, src_p, dst_p, conv3_att)
    P30, dn3 = _passB0(xh0, src_p, dst_p, e3, mx3)
    (P31,) = _passB1(xh1, src_p, dst_p, e3, mx3)

    x30, x31 = _post3(P30, P31, dn3, conv3_b)
    Pa, Pb, cnt = _pool(x30, x31, pool_idx)

    sWp = jnp.pad(sm_W, ((0, 0), (0, 2)))
    sbp = jnp.concatenate([sm_b, jnp.full((2,), -1e30, F32)]).reshape(1, -1)
    gp = jnp.concatenate([g, jnp.zeros((_BP - _B, 128), F32)], axis=0)
    emb, prob = _head(Pa, Pb, cnt, gp, linear_W, linear_b, sWp, sbp)
    return (prob[:_B, :6], emb[:_B])


# speedup vs baseline: 1.0645x; 1.0152x over previous
import functools

import jax
import jax.numpy as jnp
from jax import lax
from jax.experimental import pallas as pl
from jax.experimental.pallas import tpu as pltpu
from jax.experimental.pallas import tpu_sc as plsc

F32 = jnp.float32
I32 = jnp.int32

_N = 10000
_NP = 10240
_E = 170000
_EP = 172032
_EPT = _EP // 32
_EB = 128
_NBLK = _EPT // _EB
_B = 5000
_BP = 5120
_D = 256
_HW = 128
_RPT = _NP // 16
_PNRT = _NP // 32
_PTRT = _BP // 16

_SC_MESH = dict(core_axis_name="c", subcore_axis_name="s")
_SC_PARAMS = pltpu.CompilerParams(needs_layout_passes=False)


def _emb_body(t_ref, a_ref, gf_ref, Wt, bt, Wa, ba, Wg, bg, x1o, x2o, go):
    x1o[...] = jnp.dot(t_ref[...], Wt[...], preferred_element_type=F32) + bt[...]
    x2o[...] = jnp.dot(a_ref[...], Wa[...], preferred_element_type=F32) + ba[...]
    go[...] = jnp.dot(gf_ref[...], Wg[...], preferred_element_type=F32) + bg[...]


def _embed(type_nodes, attr_nodes, global_features, Wt, bt, Wa, ba, Wg, bg):
    return pl.pallas_call(
        _emb_body,
        out_shape=[jax.ShapeDtypeStruct((5000, 128), F32)] * 3,
    )(type_nodes, attr_nodes, global_features,
      Wt, bt.reshape(1, -1), Wa, ba.reshape(1, -1), Wg, bg.reshape(1, -1))


def _split_xl(x, Wl, Wr, xr_o, xh0_o, xh1_o):
    xl = jnp.dot(x, Wl[...], preferred_element_type=F32)
    xr_o[...] = jnp.dot(x, Wr[...], preferred_element_type=F32)
    xh0_o[...] = xl[:, :_HW]
    xh1_o[...] = xl[:, _HW:]


def _mm1_body(x_ref, Wl, Wr, xr_o, xh0_o, xh1_o):
    _split_xl(x_ref[...], Wl, Wr, xr_o, xh0_o, xh1_o)


def _combine(p0_ref, p1_ref, dn_ref, b_ref):
    acc = jnp.concatenate([p0_ref[0] + p0_ref[1], p1_ref[0] + p1_ref[1]], axis=1)
    dsum = jnp.sum(dn_ref[...], axis=0, keepdims=True)
    denom = jnp.reshape(dsum, (dsum.shape[1], 1))
    h = acc / (denom + 1e-16) + b_ref[...]
    return jnp.where(h > 0, h, 0.01 * h)


def _mm23_body(p0_ref, p1_ref, dn_ref, b_ref, Wl, Wr, xr_o, xh0_o, xh1_o):
    _split_xl(_combine(p0_ref, p1_ref, dn_ref, b_ref), Wl, Wr, xr_o, xh0_o, xh1_o)


def _post3_body(p0_ref, p1_ref, dn_ref, b_ref, x_o0, x_o1):
    x = _combine(p0_ref, p1_ref, dn_ref, b_ref)
    x_o0[...] = x[:, :_HW]
    x_o1[...] = x[:, _HW:]


_R_MM = 1280
_G_MM = _NP // _R_MM

_MM_OUT_SPECS = [
    pl.BlockSpec((_R_MM, _D), lambda i: (i, 0)),
    pl.BlockSpec((_R_MM, _HW), lambda i: (i, 0)),
    pl.BlockSpec((_R_MM, _HW), lambda i: (i, 0)),
]
_MM_OUT_SHAPE = [
    jax.ShapeDtypeStruct((_NP, _D), F32),
    jax.ShapeDtypeStruct((_NP, _HW), F32),
    jax.ShapeDtypeStruct((_NP, _HW), F32),
]


def _mm1(x0, Wl, Wr):
    return pl.pallas_call(
        _mm1_body,
        grid=(_G_MM,),
        in_specs=[
            pl.BlockSpec((_R_MM, 128), lambda i: (i, 0)),
            pl.BlockSpec((128, _D), lambda i: (0, 0)),
            pl.BlockSpec((128, _D), lambda i: (0, 0)),
        ],
        out_specs=_MM_OUT_SPECS,
        out_shape=_MM_OUT_SHAPE,
    )(x0, Wl, Wr)


_P_SPECS = [
    pl.BlockSpec((2, _R_MM, _HW), lambda i: (0, i, 0)),
    pl.BlockSpec((2, _R_MM, _HW), lambda i: (0, i, 0)),
    pl.BlockSpec((32, _R_MM), lambda i: (0, i)),
    pl.BlockSpec((1, _D), lambda i: (0, 0)),
]


def _mm23(P0, P1, dn, b, Wl, Wr):
    return pl.pallas_call(
        _mm23_body,
        grid=(_G_MM,),
        in_specs=_P_SPECS + [
            pl.BlockSpec((_D, _D), lambda i: (0, 0)),
            pl.BlockSpec((_D, _D), lambda i: (0, 0)),
        ],
        out_specs=_MM_OUT_SPECS,
        out_shape=_MM_OUT_SHAPE,
    )(P0, P1, dn, b.reshape(1, -1), Wl, Wr)


def _post3(P0, P1, dn, b):
    return pl.pallas_call(
        _post3_body,
        grid=(_G_MM,),
        in_specs=_P_SPECS,
        out_specs=[pl.BlockSpec((_R_MM, _HW), lambda i: (i, 0))] * 2,
        out_shape=[jax.ShapeDtypeStruct((_NP, _HW), F32)] * 2,
    )(P0, P1, dn, b.reshape(1, -1))


def _head_body(pa_ref, pb_ref, cnt_ref, g_ref, lW, lb, sW, sb, emb_o, prob_o):
    acc = jnp.concatenate([pa_ref[0] + pa_ref[1], pb_ref[0] + pb_ref[1]], axis=1)
    csum = jnp.sum(cnt_ref[...], axis=0, keepdims=True)
    cnt = jnp.reshape(csum, (csum.shape[1], 1))
    pooled = acc / jnp.clip(cnt, 1.0, None)
    h = jnp.concatenate([pooled, g_ref[...]], axis=1)
    emb = jnp.dot(h, lW[...], preferred_element_type=F32) + lb[...]
    emb_o[...] = emb
    lg = jnp.dot(emb, sW[...], preferred_element_type=F32) + sb[...]
    m = jnp.max(lg, axis=1, keepdims=True)
    p = jnp.exp(lg - m)
    prob_o[...] = p / jnp.sum(p, axis=1, keepdims=True)


def _head(Pa, Pb, cnt, g, lW, lb, sWp, sbp):
    return pl.pallas_call(
        _head_body,
        grid=(4,),
        in_specs=[
            pl.BlockSpec((2, 1280, _HW), lambda i: (0, i, 0)),
            pl.BlockSpec((2, 1280, _HW), lambda i: (0, i, 0)),
            pl.BlockSpec((32, 1280), lambda i: (0, i)),
            pl.BlockSpec((1280, 128), lambda i: (i, 0)),
            pl.BlockSpec((384, 128), lambda i: (0, 0)),
            pl.BlockSpec((1, 128), lambda i: (0, 0)),
            pl.BlockSpec((128, 8), lambda i: (0, 0)),
            pl.BlockSpec((1, 8), lambda i: (0, 0)),
        ],
        out_specs=[
            pl.BlockSpec((1280, 128), lambda i: (i, 0)),
            pl.BlockSpec((1280, 8), lambda i: (i, 0)),
        ],
        out_shape=[
            jax.ShapeDtypeStruct((_BP, 128), F32),
            jax.ShapeDtypeStruct((_BP, 8), F32),
        ],
    )(Pa, Pb, cnt, g, lW, lb.reshape(1, -1), sWp, sbp)


_EBA = 96
_NBA = _EPT // _EBA
_EBB = 96
_NBB = _EPT // _EBB


def _passA_body(xl0, xl1, xr, src, dst, att, e_o, mx_o,
                src_p, dst_p, bufs, e_p, att_b, acc_b, mx_b, sems):
    cid = lax.axis_index("c")
    sid = lax.axis_index("s")
    wid = sid * 2 + cid
    pltpu.sync_copy(att, att_b)
    pltpu.sync_copy(src.at[pl.ds(wid * _EPT, _EPT)], src_p)
    pltpu.sync_copy(dst.at[pl.ds(wid * _EPT, _EPT)], dst_p)
    att_v = [att_b[pl.ds(16 * k, 16)] for k in range(16)]
    zero16 = jnp.zeros((16,), F32)
    lanes = lax.iota(I32, 16)
    col_idx = [jnp.full((16,), j, I32) for j in range(16)]

    def issue(off, s):
        r0, r1, r2 = bufs[s]
        t0, t1, t2 = sems[s]
        pltpu.async_copy(xl0.at[src_p.at[pl.ds(off, _EBA)]], r0, t0)
        pltpu.async_copy(xl1.at[src_p.at[pl.ds(off, _EBA)]], r1, t1)
        pltpu.async_copy(xr.at[dst_p.at[pl.ds(off, _EBA)]], r2, t2)

    def waitset(off, s):
        r0, r1, r2 = bufs[s]
        t0, t1, t2 = sems[s]
        pltpu.make_async_copy(xl0.at[src_p.at[pl.ds(off, _EBA)]], r0, t0).wait()
        pltpu.make_async_copy(xl1.at[src_p.at[pl.ds(off, _EBA)]], r1, t1).wait()
        pltpu.make_async_copy(xr.at[dst_p.at[pl.ds(off, _EBA)]], r2, t2).wait()

    def compute(off, s, mxv):
        r0, r1, r2 = bufs[s]

        def grp(gi, mxv1):
            @plsc.parallel_loop(0, 16, unroll=2)
            def _(j):
                i = gi * 16 + j
                acc = zero16
                for k in range(8):
                    v = r0[i, pl.ds(16 * k, 16)] + r2[i, pl.ds(16 * k, 16)]
                    acc = acc + jnp.maximum(v, 0.2 * v) * att_v[k]
                for k in range(8):
                    v = r1[i, pl.ds(16 * k, 16)] + r2[i, pl.ds(128 + 16 * k, 16)]
                    acc = acc + jnp.maximum(v, 0.2 * v) * att_v[8 + k]
                acc_b[j, pl.ds(0, 16)] = acc
            e16 = plsc.load_gather(acc_b, [lanes, col_idx[0]])
            for j in range(1, 16):
                e16 = e16 + plsc.load_gather(acc_b, [lanes, col_idx[j]])
            e_p[pl.ds(off + gi * 16, 16)] = e16
            return jnp.maximum(mxv1, e16)

        return lax.fori_loop(0, _EBA // 16, grp, mxv)

    issue(0, 0)

    def outer(g, mxv):
        off_a = 2 * g * _EBA
        off_b = off_a + _EBA
        issue(off_b, 1)
        waitset(off_a, 0)
        mxv = compute(off_a, 0, mxv)

        @pl.when(2 * g + 2 < _NBA)
        def _():
            issue(off_a + 2 * _EBA, 0)

        waitset(off_b, 1)
        mxv = compute(off_b, 1, mxv)
        return mxv

    mxv = lax.fori_loop(0, _NBA // 2, outer, jnp.full((16,), F32(-3e38), F32))
    pltpu.sync_copy(e_p, e_o.at[pl.ds(wid * _EPT, _EPT)])
    mx_b[...] = mxv
    pltpu.sync_copy(mx_b, mx_o.at[wid])


@functools.partial(
    pl.kernel,
    out_type=(jax.ShapeDtypeStruct((_EP,), F32),
              jax.ShapeDtypeStruct((32, 16), F32)),
    mesh=plsc.VectorSubcoreMesh(**_SC_MESH),
    compiler_params=_SC_PARAMS,
    scratch_types=[
        pltpu.VMEM((_EPT,), I32),
        pltpu.VMEM((_EPT,), I32),
        [[pltpu.VMEM((_EBA, _HW), F32), pltpu.VMEM((_EBA, _HW), F32),
          pltpu.VMEM((_EBA, _D), F32)] for _ in range(2)],
        pltpu.VMEM((_EPT,), F32),
        pltpu.VMEM((_D,), F32),
        pltpu.VMEM((16, 16), F32),
        pltpu.VMEM((16,), F32),
        [[pltpu.SemaphoreType.DMA] * 3 for _ in range(2)],
    ],
)
def _passA(xl0, xl1, xr, src, dst, att, e_o, mx_o, *rest):
    _passA_body(xl0, xl1, xr, src, dst, att, e_o, mx_o, *rest)


def _make_passB(with_denom):
    def body(xlt, src, dst, e_in, mx_in, *rest):
        if with_denom:
            (out, dn_o, src_s, dst_b, dst_v, e_cb, rows2, mxs, zb, stage, dtab,
             table, sems) = rest
        else:
            (out, src_s, dst_b, dst_v, e_cb, rows2, mxs, zb, stage, table,
             sems) = rest
        cid = lax.axis_index("c")
        sid = lax.axis_index("s")
        wid = sid * 2 + cid
        base = wid * _EPT
        z16 = jnp.zeros((16,), F32)
        for r in range(16):
            for k in range(_HW // 16):
                zb[r, pl.ds(16 * k, 16)] = z16
        rbase = sid * _RPT

        def zloop(j, carry):
            pltpu.sync_copy(zb, table.at[pl.ds(rbase + j * 16, 16)])
            return carry

        lax.fori_loop(0, _RPT // 16, zloop, 0)
        if with_denom:
            def zdt(j, carry):
                dtab[pl.ds(j * 16, 16)] = z16
                return carry

            lax.fori_loop(0, _NP // 16, zdt, 0)
        plsc.subcore_barrier()

        pltpu.sync_copy(mx_in, mxs)
        m = mxs[0, :]
        for j in range(1, 32):
            m = jnp.maximum(m, mxs[j, :])
        M = jnp.max(m)
        lanes = lax.iota(I32, 16)
        zero16 = jnp.zeros((16,), F32)

        def issue(off, s):
            pltpu.sync_copy(src.at[pl.ds(base + off, _EBB)], src_s[s])
            pltpu.async_copy(xlt.at[src_s[s]], rows2[s], sems[s])

        def waitset(s):
            pltpu.make_async_copy(xlt.at[src_s[s]], rows2[s], sems[s]).wait()

        def consume(off, s, carry):
            rows = rows2[s]
            pltpu.sync_copy(dst.at[pl.ds(base + off, _EBB)], dst_b)
            if with_denom:
                pltpu.sync_copy(dst.at[pl.ds(base + off, _EBB)], dst_v)
            pltpu.sync_copy(e_in.at[pl.ds(base + off, _EBB)], e_cb)

            def grp(gi, c1):
                ee16 = jnp.exp(e_cb[pl.ds(16 * gi, 16)] - M)
                if with_denom:
                    dst16 = dst_v[pl.ds(16 * gi, 16)]
                    plsc.addupdate_scatter(dtab, [dst16], ee16)

                @plsc.parallel_loop(0, 16, unroll=2)
                def _(j):
                    i = gi * 16 + j
                    s2 = jnp.sum(jnp.where(lanes == j, ee16, zero16))
                    for k in range(_HW // 16):
                        rows[i, pl.ds(16 * k, 16)] = rows[i, pl.ds(16 * k, 16)] * s2

                return c1

            lax.fori_loop(0, _EBB // 16, grp, 0)
            pltpu.async_copy(rows, table.at[dst_b], sems[s], add=True).wait()
            return carry

        issue(0, 0)

        def outer(g, carry):
            off_a = 2 * g * _EBB
            off_b = off_a + _EBB
            issue(off_b, 1)
            waitset(0)
            carry = consume(off_a, 0, carry)

            @pl.when(2 * g + 2 < _NBB)
            def _():
                issue(off_a + 2 * _EBB, 0)

            waitset(1)
            carry = consume(off_b, 1, carry)
            return carry

        lax.fori_loop(0, _NBB // 2, outer, 0)
        if with_denom:
            pltpu.sync_copy(dtab, dn_o.at[wid])
        plsc.subcore_barrier()

        def dump(j, carry):
            r0 = rbase + j * 32
            pltpu.sync_copy(table.at[pl.ds(r0, 32)], stage)
            pltpu.sync_copy(stage, out.at[cid, pl.ds(r0, 32)])
            return carry

        lax.fori_loop(0, _RPT // 32, dump, 0)

    out_type = [jax.ShapeDtypeStruct((2, _NP, _HW), F32)]
    scratch = [
        [pltpu.VMEM((_EBB,), I32) for _ in range(2)],
        pltpu.VMEM((_EBB,), I32),
        pltpu.VMEM((_EBB,), I32),
        pltpu.VMEM((_EBB,), F32),
        [pltpu.VMEM((_EBB, _HW), F32) for _ in range(2)],
        pltpu.VMEM((32, 16), F32),
        pltpu.VMEM((16, _HW), F32),
        pltpu.VMEM((32, _HW), F32),
    ]
    if with_denom:
        out_type.append(jax.ShapeDtypeStruct((32, _NP), F32))
        scratch.append(pltpu.VMEM((_NP,), F32))
    scratch += [
        pltpu.VMEM_SHARED((_NP, _HW), F32),
        [pltpu.SemaphoreType.DMA for _ in range(2)],
    ]
    return pl.kernel(
        body,
        out_type=tuple(out_type),
        mesh=plsc.VectorSubcoreMesh(**_SC_MESH),
        compiler_params=_SC_PARAMS,
        scratch_types=scratch,
    )


_passB0 = _make_passB(True)
_passB1 = _make_passB(False)


def _pool_body(x30, x31, pidx, out_a, out_b, cnt_o,
               buf_a, buf_b, idx_b, idx_v, zb, stage, ctab, tab_a, tab_b,
               sem_a, sem_b):
    cid = lax.axis_index("c")
    sid = lax.axis_index("s")
    wid = sid * 2 + cid
    z16 = jnp.zeros((16,), F32)
    ones16 = jnp.ones((16,), F32)
    for r in range(16):
        for k in range(_HW // 16):
            zb[r, pl.ds(16 * k, 16)] = z16
    tbase = sid * _PTRT

    def zloop(j, carry):
        pltpu.sync_copy(zb, tab_a.at[pl.ds(tbase + j * 16, 16)])
        pltpu.sync_copy(zb, tab_b.at[pl.ds(tbase + j * 16, 16)])
        return carry

    lax.fori_loop(0, _PTRT // 16, zloop, 0)

    def zct(j, carry):
        ctab[pl.ds(j * 16, 16)] = z16
        return carry

    lax.fori_loop(0, _BP // 16, zct, 0)
    plsc.subcore_barrier()

    nbase = wid * _PNRT

    def blk(j, carry):
        r0 = nbase + j * 64
        pltpu.sync_copy(x30.at[pl.ds(r0, 64)], buf_a)
        pltpu.sync_copy(x31.at[pl.ds(r0, 64)], buf_b)
        pltpu.sync_copy(pidx.at[pl.ds(r0, 64)], idx_b)
        pltpu.sync_copy(pidx.at[pl.ds(r0, 64)], idx_v)
        for gi in range(4):
            plsc.addupdate_scatter(ctab, [idx_v[pl.ds(16 * gi, 16)]], ones16)
        ca = pltpu.async_copy(buf_a, tab_a.at[idx_b], sem_a, add=True)
        cb = pltpu.async_copy(buf_b, tab_b.at[idx_b], sem_b, add=True)
        ca.wait()
        cb.wait()
        return carry

    lax.fori_loop(0, _PNRT // 64, blk, 0)
    pltpu.sync_copy(ctab, cnt_o.at[wid])
    plsc.subcore_barrier()

    def dump(j, carry):
        r0 = tbase + j * 64
        pltpu.sync_copy(tab_a.at[pl.ds(r0, 64)], stage)
        pltpu.sync_copy(stage, out_a.at[cid, pl.ds(r0, 64)])
        pltpu.sync_copy(tab_b.at[pl.ds(r0, 64)], stage)
        pltpu.sync_copy(stage, out_b.at[cid, pl.ds(r0, 64)])
        return carry

    lax.fori_loop(0, _PTRT // 64, dump, 0)


_pool = pl.kernel(
    _pool_body,
    out_type=(jax.ShapeDtypeStruct((2, _BP, _HW), F32),
              jax.ShapeDtypeStruct((2, _BP, _HW), F32),
              jax.ShapeDtypeStruct((32, _BP), F32)),
    mesh=plsc.VectorSubcoreMesh(**_SC_MESH),
    compiler_params=_SC_PARAMS,
    scratch_types=[
        pltpu.VMEM((64, _HW), F32),
        pltpu.VMEM((64, _HW), F32),
        pltpu.VMEM((64,), I32),
        pltpu.VMEM((64,), I32),
        pltpu.VMEM((16, _HW), F32),
        pltpu.VMEM((64, _HW), F32),
        pltpu.VMEM((_BP,), F32),
        pltpu.VMEM_SHARED((_BP, _HW), F32),
        pltpu.VMEM_SHARED((_BP, _HW), F32),
        pltpu.SemaphoreType.DMA,
        pltpu.SemaphoreType.DMA,
    ],
)


def kernel(type_nodes, attr_nodes, edge_index, n_type_nodes, n_attr_nodes,
           global_features, batch_info, type_emb_W, type_emb_b, attr_emb_W,
           attr_emb_b, global_emb_W, global_emb_b, conv1_Wl, conv1_Wr,
           conv1_att, conv1_b, conv2_Wl, conv2_Wr, conv2_att, conv2_b,
           conv3_Wl, conv3_Wr, conv3_att, conv3_b, linear_W, linear_b,
           sm_W, sm_b):
    loop = jnp.arange(_N, dtype=edge_index.dtype)
    src = jnp.concatenate([edge_index[0], loop])
    dst = jnp.concatenate([edge_index[1], loop])
    zero1 = jnp.zeros((1,), n_type_nodes.dtype)
    t_off = jnp.concatenate([zero1, jnp.cumsum(n_type_nodes)[:-1]])
    a_off = 5000 + jnp.concatenate([zero1, jnp.cumsum(n_attr_nodes)[:-1]])
    gid = jnp.stack([t_off, a_off], axis=1).reshape(-1).astype(I32)
    src_p = jnp.concatenate([gid[src], jnp.zeros((_EP - _E,), I32)])
    dst_p = jnp.concatenate([gid[dst], jnp.full((_EP - _E,), _N, I32)])
    inv_gid = jnp.zeros((_N,), I32).at[gid].set(jnp.arange(_N, dtype=I32))
    pool_idx = jnp.concatenate([batch_info[inv_gid].astype(I32),
                                jnp.full((_NP - _N,), _B, I32)])

    x1, x2, g = _embed(type_nodes, attr_nodes, global_features,
                       type_emb_W, type_emb_b, attr_emb_W, attr_emb_b,
                       global_emb_W, global_emb_b)
    x0 = jnp.concatenate([x1, x2, jnp.zeros((_NP - _N, 128), F32)], axis=0)

    xr, xh0, xh1 = _mm1(x0, conv1_Wl, conv1_Wr)
    e1, mx1 = _passA(xh0, xh1, xr, src_p, dst_p, conv1_att)
    P10, dn1 = _passB0(xh0, src_p, dst_p, e1, mx1)
    (P11,) = _passB1(xh1, src_p, dst_p, e1, mx1)

    xr, xh0, xh1 = _mm23(P10, P11, dn1, conv1_b, conv2_Wl, conv2_Wr)
    e2, mx2 = _passA(xh0, xh1, xr, src_p, dst_p, conv2_att)
    P20, dn2 = _passB0(xh0, src_p, dst_p, e2, mx2)
    (P21,) = _passB1(xh1, src_p, dst_p, e2, mx2)

    xr, xh0, xh1 = _mm23(P20, P21, dn2, conv2_b, conv3_Wl, conv3_Wr)
    e3, mx3 = _passA(xh0, xh1, xr, src_p, dst_p, conv3_att)
    P30, dn3 = _passB0(xh0, src_p, dst_p, e3, mx3)
    (P31,) = _passB1(xh1, src_p, dst_p, e3, mx3)

    x30, x31 = _post3(P30, P31, dn3, conv3_b)
    Pa, Pb, cnt = _pool(x30, x31, pool_idx)

    sWp = jnp.pad(sm_W, ((0, 0), (0, 2)))
    sbp = jnp.concatenate([sm_b, jnp.full((2,), -1e30, F32)]).reshape(1, -1)
    gp = jnp.concatenate([g, jnp.zeros((_BP - _B, 128), F32)], axis=0)
    emb, prob = _head(Pa, Pb, cnt, gp, linear_W, linear_b, sWp, sbp)
    return (prob[:_B, :6], emb[:_B])
